# baseline (speedup 1.0000x reference)
import jax
import jax.numpy as jnp
from jax.experimental import pallas as pl
from jax.experimental.pallas import tpu as pltpu

LAYERS = 8
BSZ = 8
MAX_SEQ = 2048
KV_HEADS = 8
HEAD_DIM = 128
SEQLEN = 16
N_REP = 4

CHUNK = 256
S_CHUNKS = MAX_SEQ // CHUNK
LANES = KV_HEADS * HEAD_DIM
OUT_LANES = KV_HEADS * N_REP * HEAD_DIM


def _kv_kernel(scalars_ref, xk_ref, xv_ref, kc_ref, vc_ref,
               keys_ref, values_ref, kn_ref, vn_ref):
    l = pl.program_id(2)
    s = pl.program_id(1)
    layer_idx = scalars_ref[0]
    cur_pos = scalars_ref[1]

    kn_ref[...] = kc_ref[...]
    vn_ref[...] = vc_ref[...]

    off = pl.multiple_of(cur_pos - s * CHUNK, SEQLEN)

    @pl.when(l == layer_idx)
    def _():
        @pl.when((off >= 0) & (off + SEQLEN <= CHUNK))
        def _():
            kn_ref[0, 0, pl.ds(off, SEQLEN), :] = xk_ref[0]
            vn_ref[0, 0, pl.ds(off, SEQLEN), :] = xv_ref[0]

        kchunk = kn_ref[0, 0].reshape(CHUNK, KV_HEADS, 1, HEAD_DIM)
        vchunk = vn_ref[0, 0].reshape(CHUNK, KV_HEADS, 1, HEAD_DIM)
        keys_ref[0] = jnp.broadcast_to(
            kchunk, (CHUNK, KV_HEADS, N_REP, HEAD_DIM)).reshape(CHUNK, OUT_LANES)
        values_ref[0] = jnp.broadcast_to(
            vchunk, (CHUNK, KV_HEADS, N_REP, HEAD_DIM)).reshape(CHUNK, OUT_LANES)


def _run(xk, xv, k_cache, v_cache, layer_idx, cur_pos):
    kc = k_cache.reshape(LAYERS, BSZ, MAX_SEQ, LANES)
    vc = v_cache.reshape(LAYERS, BSZ, MAX_SEQ, LANES)
    xk2 = xk.reshape(BSZ, SEQLEN, LANES)
    xv2 = xv.reshape(BSZ, SEQLEN, LANES)
    scalars = jnp.array([layer_idx, cur_pos], dtype=jnp.int32)

    grid = (BSZ, S_CHUNKS, LAYERS)
    keys, values, k_new, v_new = pl.pallas_call(
        _kv_kernel,
        grid_spec=pltpu.PrefetchScalarGridSpec(
            num_scalar_prefetch=1,
            grid=grid,
            in_specs=[
                pl.BlockSpec((1, SEQLEN, LANES), lambda b, s, l, sc: (b, 0, 0)),
                pl.BlockSpec((1, SEQLEN, LANES), lambda b, s, l, sc: (b, 0, 0)),
                pl.BlockSpec((1, 1, CHUNK, LANES), lambda b, s, l, sc: (l, b, s, 0)),
                pl.BlockSpec((1, 1, CHUNK, LANES), lambda b, s, l, sc: (l, b, s, 0)),
            ],
            out_specs=[
                pl.BlockSpec((1, CHUNK, OUT_LANES), lambda b, s, l, sc: (b, s, 0)),
                pl.BlockSpec((1, CHUNK, OUT_LANES), lambda b, s, l, sc: (b, s, 0)),
                pl.BlockSpec((1, 1, CHUNK, LANES), lambda b, s, l, sc: (l, b, s, 0)),
                pl.BlockSpec((1, 1, CHUNK, LANES), lambda b, s, l, sc: (l, b, s, 0)),
            ],
        ),
        out_shape=[
            jax.ShapeDtypeStruct((BSZ, MAX_SEQ, OUT_LANES), k_cache.dtype),
            jax.ShapeDtypeStruct((BSZ, MAX_SEQ, OUT_LANES), v_cache.dtype),
            jax.ShapeDtypeStruct((LAYERS, BSZ, MAX_SEQ, LANES), k_cache.dtype),
            jax.ShapeDtypeStruct((LAYERS, BSZ, MAX_SEQ, LANES), v_cache.dtype),
        ],
    )(scalars, xk2, xv2, kc, vc)

    keys = keys.reshape(BSZ, MAX_SEQ, KV_HEADS * N_REP, HEAD_DIM)
    values = values.reshape(BSZ, MAX_SEQ, KV_HEADS * N_REP, HEAD_DIM)
    k_new = k_new.reshape(LAYERS, BSZ, MAX_SEQ, KV_HEADS, HEAD_DIM)
    v_new = v_new.reshape(LAYERS, BSZ, MAX_SEQ, KV_HEADS, HEAD_DIM)
    return keys, values, k_new, v_new


def kernel(xk, xv, k_cache, v_cache, layer_idx, cur_pos, n_rep):
    xk = xk.astype(k_cache.dtype)
    xv = xv.astype(v_cache.dtype)
    del n_rep
    return _run(xk, xv, k_cache, v_cache, layer_idx, cur_pos)


# speedup vs baseline: 2.8179x; 2.8179x over previous
import jax
import jax.numpy as jnp
from jax.experimental import pallas as pl
from jax.experimental.pallas import tpu as pltpu

LAYERS = 8
BSZ = 8
MAX_SEQ = 2048
KV_HEADS = 8
HEAD_DIM = 128
SEQLEN = 16
N_REP = 4
OUT_HEADS = KV_HEADS * N_REP

CHUNK = 256
S_CHUNKS = MAX_SEQ // CHUNK


def _kv_kernel(scalars_ref, xk_ref, xv_ref, kc_ref, vc_ref,
               keys_ref, values_ref, kn_ref, vn_ref):
    l = pl.program_id(2)
    s = pl.program_id(1)
    layer_idx = scalars_ref[0]
    cur_pos = scalars_ref[1]

    kn_ref[...] = kc_ref[...]
    vn_ref[...] = vc_ref[...]

    off = pl.multiple_of(cur_pos - s * CHUNK, SEQLEN)

    @pl.when(l == layer_idx)
    def _():
        @pl.when((off >= 0) & (off + SEQLEN <= CHUNK))
        def _():
            kn_ref[0, 0, pl.ds(off, SEQLEN), :, :] = xk_ref[0]
            vn_ref[0, 0, pl.ds(off, SEQLEN), :, :] = xv_ref[0]

        kchunk = kn_ref[0, 0].reshape(CHUNK, KV_HEADS, 1, HEAD_DIM)
        vchunk = vn_ref[0, 0].reshape(CHUNK, KV_HEADS, 1, HEAD_DIM)
        keys_ref[0] = jnp.broadcast_to(
            kchunk, (CHUNK, KV_HEADS, N_REP, HEAD_DIM)).reshape(
                CHUNK, OUT_HEADS, HEAD_DIM)
        values_ref[0] = jnp.broadcast_to(
            vchunk, (CHUNK, KV_HEADS, N_REP, HEAD_DIM)).reshape(
                CHUNK, OUT_HEADS, HEAD_DIM)


def kernel(xk, xv, k_cache, v_cache, layer_idx, cur_pos, n_rep):
    xk = xk.astype(k_cache.dtype)
    xv = xv.astype(v_cache.dtype)
    del n_rep
    scalars = jnp.array([layer_idx, cur_pos], dtype=jnp.int32)

    grid = (BSZ, S_CHUNKS, LAYERS)
    keys, values, k_new, v_new = pl.pallas_call(
        _kv_kernel,
        grid_spec=pltpu.PrefetchScalarGridSpec(
            num_scalar_prefetch=1,
            grid=grid,
            in_specs=[
                pl.BlockSpec((1, SEQLEN, KV_HEADS, HEAD_DIM),
                             lambda b, s, l, sc: (b, 0, 0, 0)),
                pl.BlockSpec((1, SEQLEN, KV_HEADS, HEAD_DIM),
                             lambda b, s, l, sc: (b, 0, 0, 0)),
                pl.BlockSpec((1, 1, CHUNK, KV_HEADS, HEAD_DIM),
                             lambda b, s, l, sc: (l, b, s, 0, 0)),
                pl.BlockSpec((1, 1, CHUNK, KV_HEADS, HEAD_DIM),
                             lambda b, s, l, sc: (l, b, s, 0, 0)),
            ],
            out_specs=[
                pl.BlockSpec((1, CHUNK, OUT_HEADS, HEAD_DIM),
                             lambda b, s, l, sc: (b, s, 0, 0)),
                pl.BlockSpec((1, CHUNK, OUT_HEADS, HEAD_DIM),
                             lambda b, s, l, sc: (b, s, 0, 0)),
                pl.BlockSpec((1, 1, CHUNK, KV_HEADS, HEAD_DIM),
                             lambda b, s, l, sc: (l, b, s, 0, 0)),
                pl.BlockSpec((1, 1, CHUNK, KV_HEADS, HEAD_DIM),
                             lambda b, s, l, sc: (l, b, s, 0, 0)),
            ],
        ),
        out_shape=[
            jax.ShapeDtypeStruct((BSZ, MAX_SEQ, OUT_HEADS, HEAD_DIM), k_cache.dtype),
            jax.ShapeDtypeStruct((BSZ, MAX_SEQ, OUT_HEADS, HEAD_DIM), v_cache.dtype),
            jax.ShapeDtypeStruct((LAYERS, BSZ, MAX_SEQ, KV_HEADS, HEAD_DIM), k_cache.dtype),
            jax.ShapeDtypeStruct((LAYERS, BSZ, MAX_SEQ, KV_HEADS, HEAD_DIM), v_cache.dtype),
        ],
    )(scalars, xk, xv, k_cache, v_cache)
    return keys, values, k_new, v_new


# speedup vs baseline: 3.4453x; 1.2226x over previous
import jax
import jax.numpy as jnp
from jax.experimental import pallas as pl
from jax.experimental.pallas import tpu as pltpu

LAYERS = 8
BSZ = 8
MAX_SEQ = 2048
KV_HEADS = 8
HEAD_DIM = 128
SEQLEN = 16
N_REP = 4
OUT_HEADS = KV_HEADS * N_REP

CHUNK = 512
S_CHUNKS = MAX_SEQ // CHUNK


def _kv_kernel(scalars_ref, xk_ref, xv_ref, kc_ref, vc_ref,
               keys_ref, values_ref, kn_ref, vn_ref):
    l = pl.program_id(2)
    s = pl.program_id(1)
    layer_idx = scalars_ref[0]
    cur_pos = scalars_ref[1]

    kn_ref[...] = kc_ref[...]
    vn_ref[...] = vc_ref[...]

    off = pl.multiple_of(cur_pos - s * CHUNK, SEQLEN)

    @pl.when(l == layer_idx)
    def _():
        @pl.when((off >= 0) & (off + SEQLEN <= CHUNK))
        def _():
            kn_ref[0, 0, pl.ds(off, SEQLEN), :, :] = xk_ref[0]
            vn_ref[0, 0, pl.ds(off, SEQLEN), :, :] = xv_ref[0]

        kchunk = kn_ref[0, 0].reshape(CHUNK, KV_HEADS, 1, HEAD_DIM)
        vchunk = vn_ref[0, 0].reshape(CHUNK, KV_HEADS, 1, HEAD_DIM)
        keys_ref[0] = jnp.broadcast_to(
            kchunk, (CHUNK, KV_HEADS, N_REP, HEAD_DIM)).reshape(
                CHUNK, OUT_HEADS, HEAD_DIM)
        values_ref[0] = jnp.broadcast_to(
            vchunk, (CHUNK, KV_HEADS, N_REP, HEAD_DIM)).reshape(
                CHUNK, OUT_HEADS, HEAD_DIM)


def kernel(xk, xv, k_cache, v_cache, layer_idx, cur_pos, n_rep):
    xk = xk.astype(k_cache.dtype)
    xv = xv.astype(v_cache.dtype)
    del n_rep
    scalars = jnp.array([layer_idx, cur_pos], dtype=jnp.int32)

    grid = (BSZ, S_CHUNKS, LAYERS)
    keys, values, k_new, v_new = pl.pallas_call(
        _kv_kernel,
        grid_spec=pltpu.PrefetchScalarGridSpec(
            num_scalar_prefetch=1,
            grid=grid,
            in_specs=[
                pl.BlockSpec((1, SEQLEN, KV_HEADS, HEAD_DIM),
                             lambda b, s, l, sc: (b, 0, 0, 0)),
                pl.BlockSpec((1, SEQLEN, KV_HEADS, HEAD_DIM),
                             lambda b, s, l, sc: (b, 0, 0, 0)),
                pl.BlockSpec((1, 1, CHUNK, KV_HEADS, HEAD_DIM),
                             lambda b, s, l, sc: (l, b, s, 0, 0)),
                pl.BlockSpec((1, 1, CHUNK, KV_HEADS, HEAD_DIM),
                             lambda b, s, l, sc: (l, b, s, 0, 0)),
            ],
            out_specs=[
                pl.BlockSpec((1, CHUNK, OUT_HEADS, HEAD_DIM),
                             lambda b, s, l, sc: (b, s, 0, 0)),
                pl.BlockSpec((1, CHUNK, OUT_HEADS, HEAD_DIM),
                             lambda b, s, l, sc: (b, s, 0, 0)),
                pl.BlockSpec((1, 1, CHUNK, KV_HEADS, HEAD_DIM),
                             lambda b, s, l, sc: (l, b, s, 0, 0)),
                pl.BlockSpec((1, 1, CHUNK, KV_HEADS, HEAD_DIM),
                             lambda b, s, l, sc: (l, b, s, 0, 0)),
            ],
        ),
        compiler_params=pltpu.CompilerParams(
            dimension_semantics=("parallel", "parallel", "arbitrary"),
        ),
        out_shape=[
            jax.ShapeDtypeStruct((BSZ, MAX_SEQ, OUT_HEADS, HEAD_DIM), k_cache.dtype),
            jax.ShapeDtypeStruct((BSZ, MAX_SEQ, OUT_HEADS, HEAD_DIM), v_cache.dtype),
            jax.ShapeDtypeStruct((LAYERS, BSZ, MAX_SEQ, KV_HEADS, HEAD_DIM), k_cache.dtype),
            jax.ShapeDtypeStruct((LAYERS, BSZ, MAX_SEQ, KV_HEADS, HEAD_DIM), v_cache.dtype),
        ],
    )(scalars, xk, xv, k_cache, v_cache)
    return keys, values, k_new, v_new


# speedup vs baseline: 3.6781x; 1.0676x over previous
import jax
import jax.numpy as jnp
from jax.experimental import pallas as pl
from jax.experimental.pallas import tpu as pltpu

LAYERS = 8
BSZ = 8
MAX_SEQ = 2048
KV_HEADS = 8
HEAD_DIM = 128
SEQLEN = 16
N_REP = 4
OUT_HEADS = KV_HEADS * N_REP

CHUNK = 1024
S_CHUNKS = MAX_SEQ // CHUNK


def _kv_kernel(scalars_ref, xk_ref, xv_ref, kc_ref, vc_ref,
               keys_ref, values_ref, kn_ref, vn_ref):
    l = pl.program_id(2)
    s = pl.program_id(1)
    layer_idx = scalars_ref[0]
    cur_pos = scalars_ref[1]

    kn_ref[...] = kc_ref[...]
    vn_ref[...] = vc_ref[...]

    off = pl.multiple_of(cur_pos - s * CHUNK, SEQLEN)

    @pl.when(l == layer_idx)
    def _():
        @pl.when((off >= 0) & (off + SEQLEN <= CHUNK))
        def _():
            kn_ref[0, 0, pl.ds(off, SEQLEN), :, :] = xk_ref[0]
            vn_ref[0, 0, pl.ds(off, SEQLEN), :, :] = xv_ref[0]

        kchunk = kn_ref[0, 0].reshape(CHUNK, KV_HEADS, 1, HEAD_DIM)
        vchunk = vn_ref[0, 0].reshape(CHUNK, KV_HEADS, 1, HEAD_DIM)
        keys_ref[0] = jnp.broadcast_to(
            kchunk, (CHUNK, KV_HEADS, N_REP, HEAD_DIM)).reshape(
                CHUNK, OUT_HEADS, HEAD_DIM)
        values_ref[0] = jnp.broadcast_to(
            vchunk, (CHUNK, KV_HEADS, N_REP, HEAD_DIM)).reshape(
                CHUNK, OUT_HEADS, HEAD_DIM)


def kernel(xk, xv, k_cache, v_cache, layer_idx, cur_pos, n_rep):
    xk = xk.astype(k_cache.dtype)
    xv = xv.astype(v_cache.dtype)
    del n_rep
    scalars = jnp.array([layer_idx, cur_pos], dtype=jnp.int32)

    grid = (BSZ, S_CHUNKS, LAYERS)
    keys, values, k_new, v_new = pl.pallas_call(
        _kv_kernel,
        grid_spec=pltpu.PrefetchScalarGridSpec(
            num_scalar_prefetch=1,
            grid=grid,
            in_specs=[
                pl.BlockSpec((1, SEQLEN, KV_HEADS, HEAD_DIM),
                             lambda b, s, l, sc: (b, 0, 0, 0)),
                pl.BlockSpec((1, SEQLEN, KV_HEADS, HEAD_DIM),
                             lambda b, s, l, sc: (b, 0, 0, 0)),
                pl.BlockSpec((1, 1, CHUNK, KV_HEADS, HEAD_DIM),
                             lambda b, s, l, sc: (l, b, s, 0, 0)),
                pl.BlockSpec((1, 1, CHUNK, KV_HEADS, HEAD_DIM),
                             lambda b, s, l, sc: (l, b, s, 0, 0)),
            ],
            out_specs=[
                pl.BlockSpec((1, CHUNK, OUT_HEADS, HEAD_DIM),
                             lambda b, s, l, sc: (b, s, 0, 0)),
                pl.BlockSpec((1, CHUNK, OUT_HEADS, HEAD_DIM),
                             lambda b, s, l, sc: (b, s, 0, 0)),
                pl.BlockSpec((1, 1, CHUNK, KV_HEADS, HEAD_DIM),
                             lambda b, s, l, sc: (l, b, s, 0, 0)),
                pl.BlockSpec((1, 1, CHUNK, KV_HEADS, HEAD_DIM),
                             lambda b, s, l, sc: (l, b, s, 0, 0)),
            ],
        ),
        compiler_params=pltpu.CompilerParams(
            dimension_semantics=("parallel", "parallel", "arbitrary"),
        ),
        out_shape=[
            jax.ShapeDtypeStruct((BSZ, MAX_SEQ, OUT_HEADS, HEAD_DIM), k_cache.dtype),
            jax.ShapeDtypeStruct((BSZ, MAX_SEQ, OUT_HEADS, HEAD_DIM), v_cache.dtype),
            jax.ShapeDtypeStruct((LAYERS, BSZ, MAX_SEQ, KV_HEADS, HEAD_DIM), k_cache.dtype),
            jax.ShapeDtypeStruct((LAYERS, BSZ, MAX_SEQ, KV_HEADS, HEAD_DIM), v_cache.dtype),
        ],
    )(scalars, xk, xv, k_cache, v_cache)
    return keys, values, k_new, v_new


# speedup vs baseline: 3.9547x; 1.0752x over previous
import jax
import jax.numpy as jnp
from jax.experimental import pallas as pl
from jax.experimental.pallas import tpu as pltpu

LAYERS = 8
BSZ = 8
MAX_SEQ = 2048
KV_HEADS = 8
HEAD_DIM = 128
SEQLEN = 16
N_REP = 4
OUT_HEADS = KV_HEADS * N_REP

CHUNK = 1024
S_CHUNKS = MAX_SEQ // CHUNK


def _kv_kernel(scalars_ref, xk_ref, xv_ref, kc_ref, vc_ref,
               keys_ref, values_ref, kn_ref, vn_ref):
    l = pl.program_id(2)
    s = pl.program_id(1)
    layer_idx = scalars_ref[0]
    cur_pos = scalars_ref[1]

    kn_ref[...] = kc_ref[...]
    vn_ref[...] = vc_ref[...]

    off = pl.multiple_of(cur_pos - s * CHUNK, SEQLEN)

    @pl.when(l == layer_idx)
    def _():
        @pl.when((off >= 0) & (off + SEQLEN <= CHUNK))
        def _():
            kn_ref[0, 0, pl.ds(off, SEQLEN), :, :] = xk_ref[0]
            vn_ref[0, 0, pl.ds(off, SEQLEN), :, :] = xv_ref[0]

        for h in range(KV_HEADS):
            ksrc = kn_ref[0, 0, :, h, :]
            vsrc = vn_ref[0, 0, :, h, :]
            keys_ref[0, :, N_REP * h:N_REP * (h + 1), :] = jnp.broadcast_to(
                ksrc[:, None, :], (CHUNK, N_REP, HEAD_DIM))
            values_ref[0, :, N_REP * h:N_REP * (h + 1), :] = jnp.broadcast_to(
                vsrc[:, None, :], (CHUNK, N_REP, HEAD_DIM))


def kernel(xk, xv, k_cache, v_cache, layer_idx, cur_pos, n_rep):
    xk = xk.astype(k_cache.dtype)
    xv = xv.astype(v_cache.dtype)
    del n_rep
    scalars = jnp.array([layer_idx, cur_pos], dtype=jnp.int32)

    grid = (BSZ, S_CHUNKS, LAYERS)
    keys, values, k_new, v_new = pl.pallas_call(
        _kv_kernel,
        grid_spec=pltpu.PrefetchScalarGridSpec(
            num_scalar_prefetch=1,
            grid=grid,
            in_specs=[
                pl.BlockSpec((1, SEQLEN, KV_HEADS, HEAD_DIM),
                             lambda b, s, l, sc: (b, 0, 0, 0)),
                pl.BlockSpec((1, SEQLEN, KV_HEADS, HEAD_DIM),
                             lambda b, s, l, sc: (b, 0, 0, 0)),
                pl.BlockSpec((1, 1, CHUNK, KV_HEADS, HEAD_DIM),
                             lambda b, s, l, sc: (l, b, s, 0, 0)),
                pl.BlockSpec((1, 1, CHUNK, KV_HEADS, HEAD_DIM),
                             lambda b, s, l, sc: (l, b, s, 0, 0)),
            ],
            out_specs=[
                pl.BlockSpec((1, CHUNK, OUT_HEADS, HEAD_DIM),
                             lambda b, s, l, sc: (b, s, 0, 0)),
                pl.BlockSpec((1, CHUNK, OUT_HEADS, HEAD_DIM),
                             lambda b, s, l, sc: (b, s, 0, 0)),
                pl.BlockSpec((1, 1, CHUNK, KV_HEADS, HEAD_DIM),
                             lambda b, s, l, sc: (l, b, s, 0, 0)),
                pl.BlockSpec((1, 1, CHUNK, KV_HEADS, HEAD_DIM),
                             lambda b, s, l, sc: (l, b, s, 0, 0)),
            ],
        ),
        compiler_params=pltpu.CompilerParams(
            dimension_semantics=("parallel", "parallel", "arbitrary"),
            vmem_limit_bytes=100 * 1024 * 1024,
        ),
        out_shape=[
            jax.ShapeDtypeStruct((BSZ, MAX_SEQ, OUT_HEADS, HEAD_DIM), k_cache.dtype),
            jax.ShapeDtypeStruct((BSZ, MAX_SEQ, OUT_HEADS, HEAD_DIM), v_cache.dtype),
            jax.ShapeDtypeStruct((LAYERS, BSZ, MAX_SEQ, KV_HEADS, HEAD_DIM), k_cache.dtype),
            jax.ShapeDtypeStruct((LAYERS, BSZ, MAX_SEQ, KV_HEADS, HEAD_DIM), v_cache.dtype),
        ],
    )(scalars, xk, xv, k_cache, v_cache)
    return keys, values, k_new, v_new
